# Initial kernel scaffold; baseline (speedup 1.0000x reference)
import jax
import jax.numpy as jnp
from jax.experimental import pallas as pl


def kernel(z, W):
    raise NotImplementedError("write your pallas kernel here")


# speedup vs baseline: 1.7725x; 1.7725x over previous
import jax
import jax.numpy as jnp
from jax.experimental import pallas as pl
from jax.experimental.pallas import tpu as pltpu

_N_E = 1024
_E_DIM = 256
_BETA = 0.25
_TOK_TILE = 256


def _vq_tile_kernel(z_ref, w_ref, quant_ref, idx_ref, loss_ref):
    zt = z_ref[0]
    zf = zt.T
    wv = w_ref[...]

    z2 = jnp.sum(zf * zf, axis=1, keepdims=True)
    w2 = jnp.sum(wv * wv, axis=1)
    mm = jnp.dot(zf, wv.T, preferred_element_type=jnp.float32)
    d = (z2 + w2[None, :]) - 2.0 * mm

    dmin = jnp.min(d, axis=1, keepdims=True)
    cols = jax.lax.broadcasted_iota(jnp.int32, d.shape, 1)
    idx = jnp.min(jnp.where(d == dmin, cols, _N_E), axis=1)

    onehot = (cols == idx[:, None]).astype(jnp.float32)
    zq = jnp.dot(onehot, wv, preferred_element_type=jnp.float32)

    diff = zq - zf
    part = jnp.sum(diff * diff, axis=0).reshape(2, 128)
    i, j = pl.program_id(0), pl.program_id(1)

    @pl.when((i == 0) & (j == 0))
    def _init():
        loss_ref[...] = jnp.zeros_like(loss_ref)

    loss_ref[0, :] += part[0] + part[1]

    quant_ref[0] = zq.T
    idx_ref[0, 0] = idx


def kernel(z, W):
    b, c, h, w = z.shape
    hw = h * w
    tiles_per_b = hw // _TOK_TILE
    z3 = z.reshape(b, c, hw)

    grid = (b, tiles_per_b)
    quant3, idx3, lossvec = pl.pallas_call(
        _vq_tile_kernel,
        grid=grid,
        in_specs=[
            pl.BlockSpec((1, _E_DIM, _TOK_TILE), lambda i, j: (i, 0, j)),
            pl.BlockSpec((_N_E, _E_DIM), lambda i, j: (0, 0)),
        ],
        out_specs=[
            pl.BlockSpec((1, _E_DIM, _TOK_TILE), lambda i, j: (i, 0, j)),
            pl.BlockSpec((1, 1, _TOK_TILE), lambda i, j: (i * tiles_per_b + j, 0, 0)),
            pl.BlockSpec((1, 128), lambda i, j: (0, 0)),
        ],
        out_shape=[
            jax.ShapeDtypeStruct((b, c, hw), jnp.float32),
            jax.ShapeDtypeStruct((b * tiles_per_b, 1, _TOK_TILE), jnp.int32),
            jax.ShapeDtypeStruct((1, 128), jnp.float32),
        ],
        compiler_params=pltpu.CompilerParams(
            dimension_semantics=("arbitrary", "arbitrary")),
    )(z3, W)

    quantize = quant3.reshape(b, c, h, w)
    index = idx3.reshape(b, h, w)
    m = jnp.sum(lossvec) / (b * hw * c)
    loss = m + _BETA * m
    return quantize, loss, index


# speedup vs baseline: 2.0090x; 1.1334x over previous
import jax
import jax.numpy as jnp
from jax.experimental import pallas as pl
from jax.experimental.pallas import tpu as pltpu

_N_E = 1024
_E_DIM = 256
_BETA = 0.25
_TOK_TILE = 512


def _vq_tile_kernel(z_ref, w_ref, quant_ref, idx_ref, loss_ref, w2_ref):
    zt = z_ref[0]
    zf = zt.T
    wv = w_ref[...]
    i, j = pl.program_id(0), pl.program_id(1)

    @pl.when((i == 0) & (j == 0))
    def _w2_once():
        w2_ref[...] = jnp.sum(wv * wv, axis=1).reshape(1, _N_E)

    z2 = jnp.sum(zf * zf, axis=1, keepdims=True)
    mm = jnp.dot(zf, wv.T, preferred_element_type=jnp.float32)
    d = (z2 + w2_ref[...]) - 2.0 * mm

    dmin = jnp.min(d, axis=1, keepdims=True)
    cols = jax.lax.broadcasted_iota(jnp.int32, d.shape, 1)
    idx = jnp.min(jnp.where(d == dmin, cols, _N_E), axis=1)

    onehot = (cols == idx[:, None]).astype(jnp.float32)
    zq = jnp.dot(onehot, wv, preferred_element_type=jnp.float32)

    diff = zq - zf
    part = jnp.sum(diff * diff, axis=0).reshape(2, 128)

    @pl.when((i == 0) & (j == 0))
    def _init():
        loss_ref[...] = jnp.zeros_like(loss_ref)

    loss_ref[0, :] += part[0] + part[1]

    quant_ref[0] = zq.T
    idx_ref[0, 0] = idx


def kernel(z, W):
    b, c, h, w = z.shape
    hw = h * w
    tiles_per_b = hw // _TOK_TILE
    z3 = z.reshape(b, c, hw)

    grid = (b, tiles_per_b)
    quant3, idx3, lossvec = pl.pallas_call(
        _vq_tile_kernel,
        grid=grid,
        in_specs=[
            pl.BlockSpec((1, _E_DIM, _TOK_TILE), lambda i, j: (i, 0, j)),
            pl.BlockSpec((_N_E, _E_DIM), lambda i, j: (0, 0)),
        ],
        out_specs=[
            pl.BlockSpec((1, _E_DIM, _TOK_TILE), lambda i, j: (i, 0, j)),
            pl.BlockSpec((1, 1, _TOK_TILE), lambda i, j: (i * tiles_per_b + j, 0, 0)),
            pl.BlockSpec((1, 128), lambda i, j: (0, 0)),
        ],
        out_shape=[
            jax.ShapeDtypeStruct((b, c, hw), jnp.float32),
            jax.ShapeDtypeStruct((b * tiles_per_b, 1, _TOK_TILE), jnp.int32),
            jax.ShapeDtypeStruct((1, 128), jnp.float32),
        ],
        scratch_shapes=[pltpu.VMEM((1, _N_E), jnp.float32)],
        compiler_params=pltpu.CompilerParams(
            dimension_semantics=("arbitrary", "arbitrary")),
    )(z3, W)

    quantize = quant3.reshape(b, c, h, w)
    index = idx3.reshape(b, h, w)
    m = jnp.sum(lossvec) / (b * hw * c)
    loss = m + _BETA * m
    return quantize, loss, index


# speedup vs baseline: 2.2166x; 1.1034x over previous
import jax
import jax.numpy as jnp
from jax.experimental import pallas as pl
from jax.experimental.pallas import tpu as pltpu

_N_E = 1024
_E_DIM = 256
_BETA = 0.25
_TOK_TILE = 1024


def _vq_tile_kernel(z_ref, w_ref, quant_ref, idx_ref, loss_ref, w2_ref):
    zt = z_ref[0]
    zf = zt.T
    wv = w_ref[...]
    i, j = pl.program_id(0), pl.program_id(1)

    @pl.when((i == 0) & (j == 0))
    def _w2_once():
        w2_ref[...] = jnp.sum(wv * wv, axis=1).reshape(1, _N_E)

    z2 = jnp.sum(zf * zf, axis=1, keepdims=True)
    mm = jnp.dot(zf, wv.T, preferred_element_type=jnp.float32)
    d = (z2 + w2_ref[...]) - 2.0 * mm

    dmin = jnp.min(d, axis=1, keepdims=True)
    cols = jax.lax.broadcasted_iota(jnp.int32, d.shape, 1)
    idx = jnp.min(jnp.where(d == dmin, cols, _N_E), axis=1)

    onehot = (cols == idx[:, None]).astype(jnp.float32)
    zq = jnp.dot(onehot, wv, preferred_element_type=jnp.float32)

    diff = zq - zf
    part = jnp.sum(diff * diff, axis=0).reshape(2, 128)

    @pl.when((i == 0) & (j == 0))
    def _init():
        loss_ref[...] = jnp.zeros_like(loss_ref)

    loss_ref[0, :] += part[0] + part[1]

    quant_ref[0] = zq.T
    idx_ref[0, 0] = idx


def kernel(z, W):
    b, c, h, w = z.shape
    hw = h * w
    tiles_per_b = hw // _TOK_TILE
    z3 = z.reshape(b, c, hw)

    grid = (b, tiles_per_b)
    quant3, idx3, lossvec = pl.pallas_call(
        _vq_tile_kernel,
        grid=grid,
        in_specs=[
            pl.BlockSpec((1, _E_DIM, _TOK_TILE), lambda i, j: (i, 0, j)),
            pl.BlockSpec((_N_E, _E_DIM), lambda i, j: (0, 0)),
        ],
        out_specs=[
            pl.BlockSpec((1, _E_DIM, _TOK_TILE), lambda i, j: (i, 0, j)),
            pl.BlockSpec((1, 1, _TOK_TILE), lambda i, j: (i * tiles_per_b + j, 0, 0)),
            pl.BlockSpec((1, 128), lambda i, j: (0, 0)),
        ],
        out_shape=[
            jax.ShapeDtypeStruct((b, c, hw), jnp.float32),
            jax.ShapeDtypeStruct((b * tiles_per_b, 1, _TOK_TILE), jnp.int32),
            jax.ShapeDtypeStruct((1, 128), jnp.float32),
        ],
        scratch_shapes=[pltpu.VMEM((1, _N_E), jnp.float32)],
        compiler_params=pltpu.CompilerParams(
            dimension_semantics=("arbitrary", "arbitrary")),
    )(z3, W)

    quantize = quant3.reshape(b, c, h, w)
    index = idx3.reshape(b, h, w)
    m = jnp.sum(lossvec) / (b * hw * c)
    loss = m + _BETA * m
    return quantize, loss, index


# speedup vs baseline: 2.3040x; 1.0394x over previous
import jax
import jax.numpy as jnp
from jax.experimental import pallas as pl
from jax.experimental.pallas import tpu as pltpu

_N_E = 1024
_E_DIM = 256
_BETA = 0.25
_TOK_TILE = 1024


def _vq_tile_kernel(z_ref, w_ref, quant_ref, idx_ref, loss_ref, w2_ref):
    zt = z_ref[0]
    zf = zt.T
    wv = w_ref[...]
    i, j = pl.program_id(0), pl.program_id(1)

    @pl.when((i == 0) & (j == 0))
    def _w2_once():
        w2_ref[...] = jnp.sum(wv * wv, axis=1).reshape(1, _N_E)

    z2 = jnp.sum(zf * zf, axis=1, keepdims=True)
    mm = jnp.dot(zf, wv.T, preferred_element_type=jnp.float32)
    d = (z2 + w2_ref[...]) - 2.0 * mm

    dmin = jnp.min(d, axis=1, keepdims=True)
    colsf = jax.lax.broadcasted_iota(jnp.int32, d.shape, 1).astype(jnp.float32)
    idxf = jnp.min(jnp.where(d == dmin, colsf, jnp.float32(_N_E)), axis=1)
    idx = idxf.astype(jnp.int32)
    onehot = (colsf == idxf[:, None]).astype(jnp.float32)
    zq = jnp.dot(onehot, wv, preferred_element_type=jnp.float32)

    diff = zq - zf
    part = jnp.sum(diff * diff, axis=0).reshape(2, 128)

    @pl.when((i == 0) & (j == 0))
    def _init():
        loss_ref[...] = jnp.zeros_like(loss_ref)

    loss_ref[0, :] += part[0] + part[1]

    quant_ref[0] = zq.T
    idx_ref[0, 0] = idx


def kernel(z, W):
    b, c, h, w = z.shape
    hw = h * w
    tiles_per_b = hw // _TOK_TILE
    z3 = z.reshape(b, c, hw)

    grid = (b, tiles_per_b)
    quant3, idx3, lossvec = pl.pallas_call(
        _vq_tile_kernel,
        grid=grid,
        in_specs=[
            pl.BlockSpec((1, _E_DIM, _TOK_TILE), lambda i, j: (i, 0, j)),
            pl.BlockSpec((_N_E, _E_DIM), lambda i, j: (0, 0)),
        ],
        out_specs=[
            pl.BlockSpec((1, _E_DIM, _TOK_TILE), lambda i, j: (i, 0, j)),
            pl.BlockSpec((1, 1, _TOK_TILE), lambda i, j: (i * tiles_per_b + j, 0, 0)),
            pl.BlockSpec((1, 128), lambda i, j: (0, 0)),
        ],
        out_shape=[
            jax.ShapeDtypeStruct((b, c, hw), jnp.float32),
            jax.ShapeDtypeStruct((b * tiles_per_b, 1, _TOK_TILE), jnp.int32),
            jax.ShapeDtypeStruct((1, 128), jnp.float32),
        ],
        scratch_shapes=[pltpu.VMEM((1, _N_E), jnp.float32)],
        compiler_params=pltpu.CompilerParams(
            dimension_semantics=("arbitrary", "arbitrary")),
    )(z3, W)

    quantize = quant3.reshape(b, c, h, w)
    index = idx3.reshape(b, h, w)
    m = jnp.sum(lossvec) / (b * hw * c)
    loss = m + _BETA * m
    return quantize, loss, index
